# baseline (speedup 1.0000x reference)
import jax
import jax.numpy as jnp
from jax import lax
from jax.experimental import pallas as pl
from jax.experimental.pallas import tpu as pltpu
from jax.experimental.pallas import tpu_sc as plsc

T = 8192
D = 2048
E = 16
K = 640
TB = 1024
NSEL = 1024
INT_MIN_I = -2147483648


def _scores_body(h_ref, e_ref, s_ref):
    s_ref[...] = lax.dot_general(e_ref[...], h_ref[...],
                                 (((1,), (1,)), ((), ())),
                                 preferred_element_type=jnp.float32)


def _sc_select_body(s_hbm, ov_hbm, oi_hbm, row_v, key_v, actA, actB, ov, oi):
    c = lax.axis_index("c")
    s = lax.axis_index("s")
    wid = s * 2 + c
    lane = lax.iota(jnp.int32, 16)
    zeros16 = jnp.zeros((16,), jnp.int32)
    U = 4

    @pl.when(wid < E)
    def _():
        pltpu.sync_copy(s_hbm.at[wid], row_v)

        def kb(i, carry):
            op_v, on_v, accp, accn = carry
            for u in range(U):
                o = (i * U + u) * 16
                v = row_v[pl.ds(o, 16)]
                b = lax.bitcast_convert_type(v, jnp.int32)
                m = jnp.where(b < 0, b ^ jnp.int32(0x7FFFFFFF), b)
                key_v[pl.ds(o, 16)] = m
                pos = m >= 0
                neg = ~pos
                b30 = ((m >> 30) & 1) == 1
                csp = plsc.cumsum(pos.astype(jnp.int32))
                plsc.store_scatter(actA, [op_v + csp - 1], o + lane, mask=pos)
                accp = accp + jnp.where(pos & b30, 1, 0).astype(jnp.int32)
                op_v = op_v + plsc.all_reduce_population_count(pos)
                csn = plsc.cumsum(neg.astype(jnp.int32))
                plsc.store_scatter(actB, [on_v + csn - 1], o + lane, mask=neg)
                accn = accn + jnp.where(neg & b30, 1, 0).astype(jnp.int32)
                on_v = on_v + plsc.all_reduce_population_count(neg)
            return (op_v, on_v, accp, accn)
        op_v, on_v, accp, accn = lax.fori_loop(
            0, T // 16 // U, kb, (zeros16, zeros16, zeros16, zeros16))
        cpos = jnp.max(op_v)

        take1 = cpos >= K
        above = jnp.where(take1, jnp.int32(0), cpos)
        prefix = jnp.where(take1, jnp.int32(INT_MIN_I), jnp.int32(0))
        n = jnp.where(take1, cpos, T - cpos)
        c1 = jnp.where(take1, jnp.sum(accp), jnp.sum(accn))

        og_v = jnp.where(take1, zeros16, zeros16 + cpos)

        def cpy(i, carry):
            msk = (i * 16 + lane) < cpos
            ii = actA[pl.ds(i * 16, 16)]
            plsc.store_scatter(oi, [i * 16 + lane], ii, mask=msk)
            return carry
        nv0 = jnp.where(take1, jnp.int32(0), (cpos + 15) // 16)
        lax.fori_loop(0, nv0, cpy, jnp.int32(0))

        bufs = [actB, actA]
        src0 = [actA, actB]
        for b in range(30, -1, -1):
            if b == 30:
                pass
            take_b = (above + c1) >= K
            want = jnp.where(take_b, jnp.int32(1), jnp.int32(0))
            above = jnp.where(take_b, above, above + c1)
            prefix = prefix | (want << b)
            nv = (n + 15) // 16

            if b == 30:
                srcs = [(actA, take1), (actB, ~take1)]
            else:
                srcs = [(bufs[(30 - b) % 2], None)]
            dst = bufs[(31 - b) % 2]

            for src, pred in srcs:
                def cpb(i, carry, n=n, src=src, dst=dst, b=b, want=want,
                        pred=pred):
                    off_v, og2, acc = carry
                    valid = (i * 16 + lane) < n
                    if pred is not None:
                        valid = valid & pred
                    idxs = src[pl.ds(i * 16, 16)]
                    mm = plsc.load_gather(key_v, [jnp.where(valid, idxs, 0)])
                    bit = (mm >> b) & 1
                    msk = valid & (bit == want)
                    cs = plsc.cumsum(msk.astype(jnp.int32))
                    plsc.store_scatter(dst, [off_v + cs - 1], idxs, mask=msk)
                    emit = valid & (bit != want) & (want == 0)
                    cse = plsc.cumsum(emit.astype(jnp.int32))
                    plsc.store_scatter(oi, [og2 + cse - 1], idxs, mask=emit)
                    og2 = og2 + plsc.all_reduce_population_count(emit)
                    nxtb = (mm >> (b - 1)) & 1 if b > 0 else bit
                    acc = acc + jnp.where(msk & (nxtb == 1), 1, 0).astype(jnp.int32)
                    return (off_v + plsc.all_reduce_population_count(msk),
                            og2, acc)
                off_v, og_v, acc = lax.fori_loop(
                    0, nv, cpb, (zeros16, og_v, zeros16))
                if pred is None or pred is take1:
                    keep_off, keep_acc = off_v, acc
                else:
                    keep_off = keep_off + off_v
                    keep_acc = keep_acc + acc
            n = jnp.max(keep_off)
            c1 = jnp.sum(keep_acc)

        take = K - above
        final_act = bufs[1]

        def tie(i, carry):
            p = i * 16 + lane
            msk = p < take
            ii = final_act[pl.ds(i * 16, 16)]
            plsc.store_scatter(oi, [above + p], ii, mask=msk)
            return carry
        lax.fori_loop(0, (take + 15) // 16, tie, jnp.int32(0))

        def pad(i, carry):
            o = K + i * 16
            ov[pl.ds(o, 16)] = jnp.full((16,), -jnp.inf, jnp.float32)
            oi[pl.ds(o, 16)] = jnp.full((16,), 0x7FFFFFF, jnp.int32)
            return carry
        lax.fori_loop(0, (NSEL - K) // 16, pad, jnp.int32(0))

        def gat(i, carry):
            for u in range(U):
                o = (i * U + u) * 16
                ii = oi[pl.ds(o, 16)]
                ov[pl.ds(o, 16)] = plsc.load_gather(row_v, [ii])
            return carry
        lax.fori_loop(0, K // 16 // U, gat, jnp.int32(0))

        pltpu.sync_copy(ov, ov_hbm.at[wid])
        pltpu.sync_copy(oi, oi_hbm.at[wid])


def _final_body(s_ref, v_ref, i_ref, ew_ref, ti_ref, cw_ref):
    v = v_ref[...]
    idx = i_ref[...]
    pos = lax.broadcasted_iota(jnp.int32, (E, NSEL), 1)

    k = 2
    while k <= NSEL:
        j = k // 2
        while j >= 1:
            up_v = jnp.roll(v, -j, 1)
            dn_v = jnp.roll(v, j, 1)
            up_i = jnp.roll(idx, -j, 1)
            dn_i = jnp.roll(idx, j, 1)
            is_lo = (pos & j) == 0
            pv = jnp.where(is_lo, up_v, dn_v)
            pi = jnp.where(is_lo, up_i, dn_i)
            desc = (pos & k) == 0
            better = (v > pv) | ((v == pv) & (idx < pi))
            keep = better == (desc == is_lo)
            v = jnp.where(keep, v, pv)
            idx = jnp.where(keep, idx, pi)
            j //= 2
        k *= 2

    top_v = v[:, :K]
    m = top_v[:, :1]
    ex = jnp.exp(top_v - m)
    denom = jnp.sum(ex, axis=1, keepdims=True)
    ew_ref[...] = ex / denom
    ti_ref[...] = idx[:, :K]

    tau = v[:, K - 1:K]
    tie = idx[:, K - 1:K]
    s = s_ref[...]
    spos = lax.broadcasted_iota(jnp.int32, (E, T), 1)
    sel = (s > tau) | ((s == tau) & (spos <= tie))
    cw_ref[...] = jnp.where(sel, jnp.exp(s - m) / denom, 0.0).T


def _normalize(x, axis=-1, eps=1e-12):
    n = jnp.linalg.norm(x, axis=axis, keepdims=True)
    return x / jnp.maximum(n, eps)


def kernel(hidden_states, expert_embeddings):
    hidden_states = _normalize(hidden_states)
    expert_embeddings = _normalize(expert_embeddings)
    scores = pl.pallas_call(
        _scores_body,
        grid=(T // TB,),
        in_specs=[pl.BlockSpec((TB, D), lambda i: (i, 0)),
                  pl.BlockSpec((E, D), lambda i: (0, 0))],
        out_specs=pl.BlockSpec((E, TB), lambda i: (0, i)),
        out_shape=jax.ShapeDtypeStruct((E, T), jnp.float32),
    )(hidden_states, expert_embeddings)

    mesh = plsc.VectorSubcoreMesh(core_axis_name="c", subcore_axis_name="s")
    sel_v, sel_i = pl.kernel(
        _sc_select_body,
        out_type=(jax.ShapeDtypeStruct((E, NSEL), jnp.float32),
                  jax.ShapeDtypeStruct((E, NSEL), jnp.int32)),
        mesh=mesh,
        compiler_params=pltpu.CompilerParams(needs_layout_passes=False),
        scratch_types=[pltpu.VMEM((T,), jnp.float32),
                       pltpu.VMEM((T,), jnp.int32),
                       pltpu.VMEM((T + 16,), jnp.int32),
                       pltpu.VMEM((T + 16,), jnp.int32),
                       pltpu.VMEM((NSEL,), jnp.float32),
                       pltpu.VMEM((NSEL,), jnp.int32)],
    )(scores)

    ew, ti, cw_t = pl.pallas_call(
        _final_body,
        out_shape=(jax.ShapeDtypeStruct((E, K), jnp.float32),
                   jax.ShapeDtypeStruct((E, K), jnp.int32),
                   jax.ShapeDtypeStruct((T, E), jnp.float32)),
    )(scores, sel_v, sel_i)
    return (ew[..., None], ti, cw_t)


# speedup vs baseline: 1.1249x; 1.1249x over previous
import jax
import jax.numpy as jnp
from jax import lax
from jax.experimental import pallas as pl
from jax.experimental.pallas import tpu as pltpu
from jax.experimental.pallas import tpu_sc as plsc

T = 8192
D = 2048
E = 16
K = 640
TB = 1024
NSEL = 1024
INT_MIN_I = -2147483648


def _scores_body(h_ref, e_ref, s_ref):
    s_ref[...] = lax.dot_general(e_ref[...], h_ref[...],
                                 (((1,), (1,)), ((), ())),
                                 preferred_element_type=jnp.float32)


def _sc_select_body(s_hbm, ov_hbm, oi_hbm, row_v, key_v, act, ov, oi):
    c = lax.axis_index("c")
    s = lax.axis_index("s")
    wid = s * 2 + c
    lane = lax.iota(jnp.int32, 16)
    zeros16 = jnp.zeros((16,), jnp.int32)

    @pl.when(wid < E)
    def _():
        pltpu.sync_copy(s_hbm.at[wid], row_v)

        def kb(i, carry):
            op_v, on_v, accp, accn = carry
            for u in range(4):
                o = (i * 4 + u) * 16
                v = row_v[pl.ds(o, 16)]
                b = lax.bitcast_convert_type(v, jnp.int32)
                m = jnp.where(b < 0, b ^ jnp.int32(0x7FFFFFFF), b)
                key_v[pl.ds(o, 16)] = m
                pos = m >= 0
                neg = ~pos
                b30 = ((m >> 30) & 1) == 1
                csp = plsc.cumsum(pos.astype(jnp.int32))
                plsc.store_scatter(act, [op_v + csp - 1], o + lane, mask=pos)
                accp = accp + jnp.where(pos & b30, 1, 0).astype(jnp.int32)
                op_v = op_v + plsc.all_reduce_population_count(pos)
                csn = plsc.cumsum(neg.astype(jnp.int32))
                plsc.store_scatter(act, [(T - 1) - (on_v + csn - 1)],
                                   o + lane, mask=neg)
                accn = accn + jnp.where(neg & b30, 1, 0).astype(jnp.int32)
                on_v = on_v + plsc.all_reduce_population_count(neg)
            return (op_v, on_v, accp, accn)
        op_v, on_v, accp, accn = lax.fori_loop(
            0, T // 16 // 4, kb, (zeros16, zeros16, zeros16, zeros16))
        cpos = jnp.max(op_v)

        take1 = cpos >= K
        above = jnp.where(take1, jnp.int32(0), cpos)
        n = jnp.where(take1, cpos, T - cpos)
        c1 = jnp.where(take1, jnp.sum(accp), jnp.sum(accn))

        og_v = jnp.where(take1, zeros16, zeros16 + cpos)

        def cpy(i, carry):
            msk = (i * 16 + lane) < cpos
            ii = act[pl.ds(i * 16, 16)]
            plsc.store_scatter(oi, [i * 16 + lane], ii, mask=msk)
            return carry
        nv0 = jnp.where(take1, jnp.int32(0), (cpos + 15) // 16)
        lax.fori_loop(0, nv0, cpy, jnp.int32(0))

        src_base = jnp.where(take1, jnp.int32(0), cpos)
        for b in range(30, -1, -1):
            take_b = (above + c1) >= K
            want = jnp.where(take_b, jnp.int32(1), jnp.int32(0))
            above = jnp.where(take_b, above, above + c1)
            skip = jnp.where(take_b, c1 == n, c1 == 0)
            nv = jnp.where(skip & (src_base == 0), jnp.int32(0),
                           (n + 15) // 16)
            n_new = jnp.where(take_b, c1, n - c1)

            def cpb(i, carry, n=n, b=b, want=want, src_base=src_base):
                off_v, og2, acc = carry
                valid = (i * 16 + lane) < n
                idxs = act[pl.ds(src_base + i * 16, 16)]
                mm = plsc.load_gather(key_v, [jnp.where(valid, idxs, 0)])
                bit = (mm >> b) & 1
                msk = valid & (bit == want)
                cs = plsc.cumsum(msk.astype(jnp.int32))
                plsc.store_scatter(act, [off_v + cs - 1], idxs, mask=msk)
                emit = valid & (bit != want) & (want == 0)
                cse = plsc.cumsum(emit.astype(jnp.int32))
                plsc.store_scatter(oi, [og2 + cse - 1], idxs, mask=emit)
                og2 = og2 + plsc.all_reduce_population_count(emit)
                nxtb = (mm >> (b - 1)) & 1 if b > 0 else bit
                acc = acc + jnp.where(msk & (nxtb == 1), 1, 0).astype(jnp.int32)
                return (off_v + plsc.all_reduce_population_count(msk),
                        og2, acc)
            off_v, og_v, acc = lax.fori_loop(
                0, nv, cpb, (zeros16, og_v, zeros16))
            def cnt(i, acc2, b=b, n=n, src_base=src_base):
                valid = (i * 16 + lane) < n
                idxs = act[pl.ds(src_base + i * 16, 16)]
                mm = plsc.load_gather(key_v, [jnp.where(valid, idxs, 0)])
                nxtb = (mm >> (b - 1)) & 1 if b > 0 else (mm >> b) & 1
                return acc2 + jnp.where(valid & (nxtb == 1), 1, 0).astype(jnp.int32)
            nv_cnt = jnp.where(skip & (src_base == 0) & (b > 0),
                               (n + 15) // 16, jnp.int32(0))
            acc_skip = lax.fori_loop(0, nv_cnt, cnt, zeros16)
            c1 = jnp.where(nv == 0, jnp.sum(acc_skip), jnp.sum(acc))
            n = jnp.where(nv == 0, n, n_new)
            src_base = jnp.int32(0)

        take = K - above
        n_f = n
        tie_base = jnp.where(take1, jnp.int32(0), n_f - take)

        def tie(i, carry):
            p = i * 16 + lane
            msk = p < take
            ii = plsc.load_gather(act, [jnp.where(msk, tie_base + p, 0)])
            plsc.store_scatter(oi, [above + p], ii, mask=msk)
            return carry
        lax.fori_loop(0, (take + 15) // 16, tie, jnp.int32(0))

        def pad(i, carry):
            o = K + i * 16
            ov[pl.ds(o, 16)] = jnp.full((16,), -jnp.inf, jnp.float32)
            oi[pl.ds(o, 16)] = jnp.full((16,), 0x7FFFFFF, jnp.int32)
            return carry
        lax.fori_loop(0, (NSEL - K) // 16, pad, jnp.int32(0))

        def gat(i, carry):
            for u in range(4):
                o = (i * 4 + u) * 16
                ii = oi[pl.ds(o, 16)]
                ov[pl.ds(o, 16)] = plsc.load_gather(row_v, [ii])
            return carry
        lax.fori_loop(0, K // 16 // 4, gat, jnp.int32(0))

        pltpu.sync_copy(ov, ov_hbm.at[wid])
        pltpu.sync_copy(oi, oi_hbm.at[wid])


def _final_body(s_ref, v_ref, i_ref, ew_ref, ti_ref, cw_ref):
    v = v_ref[...]
    idx = i_ref[...]
    pos = lax.broadcasted_iota(jnp.int32, (E, NSEL), 1)

    k = 2
    while k <= NSEL:
        j = k // 2
        while j >= 1:
            up_v = jnp.roll(v, -j, 1)
            dn_v = jnp.roll(v, j, 1)
            up_i = jnp.roll(idx, -j, 1)
            dn_i = jnp.roll(idx, j, 1)
            is_lo = (pos & j) == 0
            pv = jnp.where(is_lo, up_v, dn_v)
            pi = jnp.where(is_lo, up_i, dn_i)
            desc = (pos & k) == 0
            better = (v > pv) | ((v == pv) & (idx < pi))
            keep = better == (desc == is_lo)
            v = jnp.where(keep, v, pv)
            idx = jnp.where(keep, idx, pi)
            j //= 2
        k *= 2

    top_v = v[:, :K]
    m = top_v[:, :1]
    ex = jnp.exp(top_v - m)
    denom = jnp.sum(ex, axis=1, keepdims=True)
    ew_ref[...] = ex / denom
    ti_ref[...] = idx[:, :K]

    tau = v[:, K - 1:K]
    tie = idx[:, K - 1:K]
    s = s_ref[...]
    spos = lax.broadcasted_iota(jnp.int32, (E, T), 1)
    sel = (s > tau) | ((s == tau) & (spos <= tie))
    cw_ref[...] = jnp.where(sel, jnp.exp(s - m) / denom, 0.0).T


def _normalize(x, axis=-1, eps=1e-12):
    n = jnp.linalg.norm(x, axis=axis, keepdims=True)
    return x / jnp.maximum(n, eps)


def kernel(hidden_states, expert_embeddings):
    hidden_states = _normalize(hidden_states)
    expert_embeddings = _normalize(expert_embeddings)
    scores = pl.pallas_call(
        _scores_body,
        grid=(T // TB,),
        in_specs=[pl.BlockSpec((TB, D), lambda i: (i, 0)),
                  pl.BlockSpec((E, D), lambda i: (0, 0))],
        out_specs=pl.BlockSpec((E, TB), lambda i: (0, i)),
        out_shape=jax.ShapeDtypeStruct((E, T), jnp.float32),
    )(hidden_states, expert_embeddings)

    mesh = plsc.VectorSubcoreMesh(core_axis_name="c", subcore_axis_name="s")
    sel_v, sel_i = pl.kernel(
        _sc_select_body,
        out_type=(jax.ShapeDtypeStruct((E, NSEL), jnp.float32),
                  jax.ShapeDtypeStruct((E, NSEL), jnp.int32)),
        mesh=mesh,
        compiler_params=pltpu.CompilerParams(needs_layout_passes=False),
        scratch_types=[pltpu.VMEM((T,), jnp.float32),
                       pltpu.VMEM((T,), jnp.int32),
                       pltpu.VMEM((T + 16,), jnp.int32),
                       pltpu.VMEM((NSEL,), jnp.float32),
                       pltpu.VMEM((NSEL,), jnp.int32)],
    )(scores)

    ew, ti, cw_t = pl.pallas_call(
        _final_body,
        out_shape=(jax.ShapeDtypeStruct((E, K), jnp.float32),
                   jax.ShapeDtypeStruct((E, K), jnp.int32),
                   jax.ShapeDtypeStruct((T, E), jnp.float32)),
    )(scores, sel_v, sel_i)
    return (ew[..., None], ti, cw_t)
